# speedup vs baseline: 2.8611x; 2.8611x over previous
import functools

import jax
import jax.numpy as jnp
from jax import lax
from jax.experimental import pallas as pl
from jax.experimental.pallas import tpu as pltpu
from jax.experimental.pallas import tpu_sc as plsc


def _sc_gather(edge_flat, drug_emb, disease_emb):
    two_e = edge_flat.shape[0]
    e = two_e // 2
    d = drug_emb.shape[1]
    info = plsc.get_sparse_core_info()
    nc, ns = info.num_cores, info.num_subcores
    nw = nc * ns
    b_per_w = e // nw
    mesh = plsc.VectorSubcoreMesh(core_axis_name="c", subcore_axis_name="s")

    @functools.partial(
        pl.kernel,
        mesh=mesh,
        out_type=(
            jax.ShapeDtypeStruct((e, d), jnp.float32),
            jax.ShapeDtypeStruct((e, d), jnp.float32),
        ),
        scratch_types=[
            pltpu.VMEM((b_per_w,), jnp.int32),
            pltpu.VMEM((b_per_w, d), jnp.float32),
            pltpu.SemaphoreType.DMA,
        ],
    )
    def gather_kernel(drug_hbm, dis_hbm, idx_hbm, d_out, s_out, idx_v, rows_v, sem):
        wid = lax.axis_index("s") * nc + lax.axis_index("c")
        base = wid * b_per_w
        pltpu.sync_copy(idx_hbm.at[pl.ds(base, b_per_w)], idx_v)
        pltpu.async_copy(drug_hbm.at[idx_v], rows_v, sem).wait()
        pltpu.sync_copy(rows_v, d_out.at[pl.ds(base, b_per_w)])
        pltpu.sync_copy(idx_hbm.at[pl.ds(e + base, b_per_w)], idx_v)
        pltpu.async_copy(dis_hbm.at[idx_v], rows_v, sem).wait()
        pltpu.sync_copy(rows_v, s_out.at[pl.ds(base, b_per_w)])

    return gather_kernel(drug_emb, disease_emb, edge_flat)


def _mlp_block_kernel(d_ref, s_ref, w1a_ref, w1b_ref, b1_ref, w2_ref, b2_ref, o_ref):
    x = jnp.dot(d_ref[...], w1a_ref[...], preferred_element_type=jnp.float32)
    x = x + jnp.dot(s_ref[...], w1b_ref[...], preferred_element_type=jnp.float32)
    h = jax.nn.sigmoid(x + b1_ref[...])
    y = jnp.dot(h, w2_ref[...], preferred_element_type=jnp.float32) + b2_ref[...]
    o_ref[...] = jax.nn.sigmoid(y)


def _mlp(d_rows, s_rows, W1, b1, W2, b2, block_e=2048):
    e, dim = d_rows.shape
    w1a = W1[:dim]
    w1b = W1[dim:]
    b1r = b1.reshape(1, -1)
    b2r = b2.reshape(1, 1)
    grid = (e // block_e,)
    return pl.pallas_call(
        _mlp_block_kernel,
        grid=grid,
        in_specs=[
            pl.BlockSpec((block_e, dim), lambda i: (i, 0)),
            pl.BlockSpec((block_e, dim), lambda i: (i, 0)),
            pl.BlockSpec(w1a.shape, lambda i: (0, 0)),
            pl.BlockSpec(w1b.shape, lambda i: (0, 0)),
            pl.BlockSpec(b1r.shape, lambda i: (0, 0)),
            pl.BlockSpec(W2.shape, lambda i: (0, 0)),
            pl.BlockSpec(b2r.shape, lambda i: (0, 0)),
        ],
        out_specs=pl.BlockSpec((block_e, 1), lambda i: (i, 0)),
        out_shape=jax.ShapeDtypeStruct((e, 1), jnp.float32),
    )(d_rows, s_rows, w1a, w1b, b1r, W2, b2r)


def kernel(edge_index, drug_emb, disease_emb, W1, b1, W2, b2):
    edge_flat = edge_index.astype(jnp.int32).reshape(-1)
    d_rows, s_rows = _sc_gather(edge_flat, drug_emb, disease_emb)
    return _mlp(d_rows, s_rows, W1, b1, W2, b2)


# speedup vs baseline: 2.8735x; 1.0043x over previous
import functools

import jax
import jax.numpy as jnp
from jax import lax
from jax.experimental import pallas as pl
from jax.experimental.pallas import tpu as pltpu
from jax.experimental.pallas import tpu_sc as plsc


def _sc_gather(edge_flat, drug_emb, disease_emb):
    two_e = edge_flat.shape[0]
    e = two_e // 2
    d = drug_emb.shape[1]
    info = plsc.get_sparse_core_info()
    nc, ns = info.num_cores, info.num_subcores
    nw = nc * ns
    b_per_w = e // nw
    mesh = plsc.VectorSubcoreMesh(core_axis_name="c", subcore_axis_name="s")

    @functools.partial(
        pl.kernel,
        mesh=mesh,
        out_type=(
            jax.ShapeDtypeStruct((e, d), jnp.float32),
            jax.ShapeDtypeStruct((e, d), jnp.float32),
        ),
        scratch_types=[
            pltpu.VMEM((b_per_w,), jnp.int32),
            pltpu.VMEM((b_per_w, d), jnp.float32),
            pltpu.SemaphoreType.DMA,
        ],
    )
    def gather_kernel(drug_hbm, dis_hbm, idx_hbm, d_out, s_out, idx_v, rows_v, sem):
        wid = lax.axis_index("s") * nc + lax.axis_index("c")
        base = wid * b_per_w
        pltpu.sync_copy(idx_hbm.at[pl.ds(base, b_per_w)], idx_v)
        pltpu.async_copy(drug_hbm.at[idx_v], rows_v, sem).wait()
        pltpu.sync_copy(rows_v, d_out.at[pl.ds(base, b_per_w)])
        pltpu.sync_copy(idx_hbm.at[pl.ds(e + base, b_per_w)], idx_v)
        pltpu.async_copy(dis_hbm.at[idx_v], rows_v, sem).wait()
        pltpu.sync_copy(rows_v, s_out.at[pl.ds(base, b_per_w)])

    return gather_kernel(drug_emb, disease_emb, edge_flat)


def _mlp_block_kernel(d_ref, s_ref, w1a_ref, w1b_ref, b1_ref, w2_ref, b2_ref, o_ref):
    d_bf = d_ref[...].astype(jnp.bfloat16)
    s_bf = s_ref[...].astype(jnp.bfloat16)
    x = jnp.dot(d_bf, w1a_ref[...], preferred_element_type=jnp.float32)
    x = x + jnp.dot(s_bf, w1b_ref[...], preferred_element_type=jnp.float32)
    h = jax.nn.sigmoid(x + b1_ref[...])
    y = jnp.dot(h, w2_ref[...], preferred_element_type=jnp.float32) + b2_ref[...]
    o_ref[...] = jax.nn.sigmoid(y)


def _mlp(d_rows, s_rows, W1, b1, W2, b2, block_e=2048):
    e, dim = d_rows.shape
    w1a = W1[:dim].astype(jnp.bfloat16)
    w1b = W1[dim:].astype(jnp.bfloat16)
    b1r = b1.reshape(1, -1)
    b2r = b2.reshape(1, 1)
    grid = (e // block_e,)
    return pl.pallas_call(
        _mlp_block_kernel,
        grid=grid,
        in_specs=[
            pl.BlockSpec((block_e, dim), lambda i: (i, 0)),
            pl.BlockSpec((block_e, dim), lambda i: (i, 0)),
            pl.BlockSpec(w1a.shape, lambda i: (0, 0)),
            pl.BlockSpec(w1b.shape, lambda i: (0, 0)),
            pl.BlockSpec(b1r.shape, lambda i: (0, 0)),
            pl.BlockSpec(W2.shape, lambda i: (0, 0)),
            pl.BlockSpec(b2r.shape, lambda i: (0, 0)),
        ],
        out_specs=pl.BlockSpec((block_e, 1), lambda i: (i, 0)),
        out_shape=jax.ShapeDtypeStruct((e, 1), jnp.float32),
    )(d_rows, s_rows, w1a, w1b, b1r, W2, b2r)


def kernel(edge_index, drug_emb, disease_emb, W1, b1, W2, b2):
    edge_flat = edge_index.astype(jnp.int32).reshape(-1)
    d_rows, s_rows = _sc_gather(edge_flat, drug_emb, disease_emb)
    return _mlp(d_rows, s_rows, W1, b1, W2, b2)


# speedup vs baseline: 3.1714x; 1.1037x over previous
import functools

import jax
import jax.numpy as jnp
from jax import lax
from jax.experimental import pallas as pl
from jax.experimental.pallas import tpu as pltpu
from jax.experimental.pallas import tpu_sc as plsc


def _sc_gather(edge_flat, drug_emb, disease_emb):
    two_e = edge_flat.shape[0]
    e = two_e // 2
    d = drug_emb.shape[1]
    info = plsc.get_sparse_core_info()
    nc, ns = info.num_cores, info.num_subcores
    nw = nc * ns
    b_per_w = e // nw
    mesh = plsc.VectorSubcoreMesh(core_axis_name="c", subcore_axis_name="s")

    @functools.partial(
        pl.kernel,
        mesh=mesh,
        out_type=(
            jax.ShapeDtypeStruct((e, d), jnp.float32),
            jax.ShapeDtypeStruct((e, d), jnp.float32),
        ),
        scratch_types=[
            pltpu.VMEM((b_per_w,), jnp.int32),
            pltpu.VMEM((b_per_w, d), jnp.float32),
            pltpu.SemaphoreType.DMA,
        ],
    )
    def gather_kernel(drug_hbm, dis_hbm, idx_hbm, d_out, s_out, idx_v, rows_v, sem):
        wid = lax.axis_index("s") * nc + lax.axis_index("c")
        base = wid * b_per_w
        pltpu.sync_copy(idx_hbm.at[pl.ds(base, b_per_w)], idx_v)
        pltpu.async_copy(drug_hbm.at[idx_v], rows_v, sem).wait()
        pltpu.sync_copy(rows_v, d_out.at[pl.ds(base, b_per_w)])
        pltpu.sync_copy(idx_hbm.at[pl.ds(e + base, b_per_w)], idx_v)
        pltpu.async_copy(dis_hbm.at[idx_v], rows_v, sem).wait()
        pltpu.sync_copy(rows_v, s_out.at[pl.ds(base, b_per_w)])

    return gather_kernel(drug_emb, disease_emb, edge_flat)


def _mlp_block_kernel(d_ref, s_ref, w1a_ref, w1b_ref, b1_ref, w2_ref, b2_ref, o_ref):
    d_bf = d_ref[...].astype(jnp.bfloat16)
    s_bf = s_ref[...].astype(jnp.bfloat16)
    x = jnp.dot(d_bf, w1a_ref[...], preferred_element_type=jnp.float32)
    x = x + jnp.dot(s_bf, w1b_ref[...], preferred_element_type=jnp.float32)
    h = jax.nn.sigmoid(x + b1_ref[...])
    y = jnp.dot(h, w2_ref[...], preferred_element_type=jnp.float32) + b2_ref[...]
    o_ref[...] = jax.nn.sigmoid(y[:, 0])


def _mlp(d_rows, s_rows, W1, b1, W2, b2, block_e=4096):
    e, dim = d_rows.shape
    w1a = W1[:dim].astype(jnp.bfloat16)
    w1b = W1[dim:].astype(jnp.bfloat16)
    b1r = b1.reshape(1, -1)
    b2r = b2.reshape(1, 1)
    grid = (e // block_e,)
    return pl.pallas_call(
        _mlp_block_kernel,
        grid=grid,
        in_specs=[
            pl.BlockSpec((block_e, dim), lambda i: (i, 0)),
            pl.BlockSpec((block_e, dim), lambda i: (i, 0)),
            pl.BlockSpec(w1a.shape, lambda i: (0, 0)),
            pl.BlockSpec(w1b.shape, lambda i: (0, 0)),
            pl.BlockSpec(b1r.shape, lambda i: (0, 0)),
            pl.BlockSpec(W2.shape, lambda i: (0, 0)),
            pl.BlockSpec(b2r.shape, lambda i: (0, 0)),
        ],
        out_specs=pl.BlockSpec((block_e,), lambda i: (i,)),
        out_shape=jax.ShapeDtypeStruct((e,), jnp.float32),
    )(d_rows, s_rows, w1a, w1b, b1r, W2, b2r)


def kernel(edge_index, drug_emb, disease_emb, W1, b1, W2, b2):
    edge_flat = edge_index.astype(jnp.int32).reshape(-1)
    d_rows, s_rows = _sc_gather(edge_flat, drug_emb, disease_emb)
    return _mlp(d_rows, s_rows, W1, b1, W2, b2).reshape(-1, 1)


# speedup vs baseline: 3.1809x; 1.0030x over previous
import functools

import jax
import jax.numpy as jnp
from jax import lax
from jax.experimental import pallas as pl
from jax.experimental.pallas import tpu as pltpu
from jax.experimental.pallas import tpu_sc as plsc


def _sc_gather(edge_flat, drug_emb, disease_emb):
    two_e = edge_flat.shape[0]
    e = two_e // 2
    d = drug_emb.shape[1]
    info = plsc.get_sparse_core_info()
    nc, ns = info.num_cores, info.num_subcores
    nw = nc * ns
    b_per_w = e // nw
    mesh = plsc.VectorSubcoreMesh(core_axis_name="c", subcore_axis_name="s")

    @functools.partial(
        pl.kernel,
        mesh=mesh,
        out_type=(
            jax.ShapeDtypeStruct((e, d), jnp.float32),
            jax.ShapeDtypeStruct((e, d), jnp.float32),
        ),
        scratch_types=[
            pltpu.VMEM((b_per_w,), jnp.int32),
            pltpu.VMEM((b_per_w, d), jnp.float32),
            pltpu.SemaphoreType.DMA,
        ],
    )
    def gather_kernel(drug_hbm, dis_hbm, idx_hbm, d_out, s_out, idx_v, rows_v, sem):
        wid = lax.axis_index("s") * nc + lax.axis_index("c")
        base = wid * b_per_w
        pltpu.sync_copy(idx_hbm.at[pl.ds(base, b_per_w)], idx_v)
        pltpu.async_copy(drug_hbm.at[idx_v], rows_v, sem).wait()
        pltpu.sync_copy(rows_v, d_out.at[pl.ds(base, b_per_w)])
        pltpu.sync_copy(idx_hbm.at[pl.ds(e + base, b_per_w)], idx_v)
        pltpu.async_copy(dis_hbm.at[idx_v], rows_v, sem).wait()
        pltpu.sync_copy(rows_v, s_out.at[pl.ds(base, b_per_w)])

    return gather_kernel(drug_emb, disease_emb, edge_flat)


def _sigmoid(x):
    return 0.5 * jnp.tanh(0.5 * x) + 0.5


def _mlp_block_kernel(d_ref, s_ref, w1a_ref, w1b_ref, b1_ref, w2_ref, b2_ref, o_ref):
    d_bf = d_ref[...].astype(jnp.bfloat16)
    s_bf = s_ref[...].astype(jnp.bfloat16)
    x = jnp.dot(d_bf, w1a_ref[...], preferred_element_type=jnp.float32)
    x = x + jnp.dot(s_bf, w1b_ref[...], preferred_element_type=jnp.float32)
    h = _sigmoid(x + b1_ref[...])
    y = jnp.dot(h, w2_ref[...], preferred_element_type=jnp.float32) + b2_ref[...]
    o_ref[...] = _sigmoid(y[:, 0])


def _mlp(d_rows, s_rows, W1, b1, W2, b2, block_e=4096):
    e, dim = d_rows.shape
    w1a = W1[:dim].astype(jnp.bfloat16)
    w1b = W1[dim:].astype(jnp.bfloat16)
    b1r = b1.reshape(1, -1)
    b2r = b2.reshape(1, 1)
    grid = (e // block_e,)
    return pl.pallas_call(
        _mlp_block_kernel,
        grid=grid,
        in_specs=[
            pl.BlockSpec((block_e, dim), lambda i: (i, 0)),
            pl.BlockSpec((block_e, dim), lambda i: (i, 0)),
            pl.BlockSpec(w1a.shape, lambda i: (0, 0)),
            pl.BlockSpec(w1b.shape, lambda i: (0, 0)),
            pl.BlockSpec(b1r.shape, lambda i: (0, 0)),
            pl.BlockSpec(W2.shape, lambda i: (0, 0)),
            pl.BlockSpec(b2r.shape, lambda i: (0, 0)),
        ],
        out_specs=pl.BlockSpec((block_e,), lambda i: (i,)),
        out_shape=jax.ShapeDtypeStruct((e,), jnp.float32),
        compiler_params=pltpu.CompilerParams(
            dimension_semantics=("parallel",),
        ),
    )(d_rows, s_rows, w1a, w1b, b1r, W2, b2r)


def kernel(edge_index, drug_emb, disease_emb, W1, b1, W2, b2):
    edge_flat = edge_index.astype(jnp.int32).reshape(-1)
    d_rows, s_rows = _sc_gather(edge_flat, drug_emb, disease_emb)
    return _mlp(d_rows, s_rows, W1, b1, W2, b2).reshape(-1, 1)
